# baseline (speedup 1.0000x reference)
import functools

import jax
import jax.numpy as jnp
from jax import lax
from jax.experimental import pallas as pl
from jax.experimental.pallas import tpu as pltpu
from jax.experimental.pallas import tpu_sc as plsc

SEQ_LEN = 200
BATCH = 1024
DIM = 64
PAIR_W = 128
N = SEQ_LEN * BATCH
NUM_WORKERS = 32
B_PER_W = N // NUM_WORKERS
CHUNK = 128
GATHERS_PER_FILL = 1
ROWS_PER_FILL = CHUNK * GATHERS_PER_FILL
N_FILLS = B_PER_W // ROWS_PER_FILL
ROW_GROUPS = ROWS_PER_FILL // 16


def _make_gather():
    mesh = plsc.VectorSubcoreMesh(core_axis_name="c", subcore_axis_name="s",
                                  num_cores=2)

    @functools.partial(
        pl.kernel,
        mesh=mesh,
        out_type=jax.ShapeDtypeStruct((NUM_WORKERS, N_FILLS, ROWS_PER_FILL,
                                       DIM), jnp.float32),
        scratch_types=[
            pltpu.VMEM((B_PER_W,), jnp.int32),
            pltpu.VMEM((B_PER_W,), jnp.int32),
            pltpu.VMEM((ROWS_PER_FILL, PAIR_W), jnp.float32),
            pltpu.VMEM((ROWS_PER_FILL, PAIR_W), jnp.float32),
            pltpu.VMEM((ROWS_PER_FILL, DIM), jnp.float32),
            pltpu.VMEM((ROWS_PER_FILL, DIM), jnp.float32),
            pltpu.SemaphoreType.DMA,
            pltpu.SemaphoreType.DMA,
            pltpu.SemaphoreType.DMA,
            pltpu.SemaphoreType.DMA,
        ],
        compiler_params=pltpu.CompilerParams(use_tc_tiling_on_sc=True,
                                             disable_bounds_checks=True,
                                             needs_layout_passes=False),
    )
    def gather(table_hbm, q_hbm, hoff_hbm, out_hbm, qv, hv,
               pairs0, pairs1, outb0, outb1, gsem0, gsem1, wsem0, wsem1):
        wid = lax.axis_index("s") * 2 + lax.axis_index("c")
        pltpu.sync_copy(q_hbm.at[pl.ds(wid * B_PER_W, B_PER_W)], qv)
        pltpu.sync_copy(hoff_hbm.at[pl.ds(wid * B_PER_W, B_PER_W)], hv)
        pairs = (pairs0, pairs1)
        outb = (outb0, outb1)
        gsem = (gsem0, gsem1)
        wsem = (wsem0, wsem1)
        lanes = lax.iota(jnp.int32, 16)

        def fill_and_drain(g, b):
            hs = [
                pltpu.async_copy(
                    table_hbm.at[qv.at[pl.ds(
                        g * ROWS_PER_FILL + c * CHUNK, CHUNK)]],
                    pairs[b].at[pl.ds(c * CHUNK, CHUNK)],
                    gsem[b])
                for c in range(GATHERS_PER_FILL)
            ]
            for h in hs:
                h.wait()

        def extract(g, b):
            @pl.loop(0, ROW_GROUPS)
            def _(i0):
                rows = i0 * 16 + lanes
                hvec = hv[pl.ds(g * ROWS_PER_FILL + i0 * 16, 16)]
                for c in range(DIM):
                    v = plsc.load_gather(pairs[b], [rows, hvec + c])
                    plsc.store_scatter(outb[b], [rows, lanes * 0 + c], v)

        def start_writeout(g, b):
            pltpu.async_copy(outb[b], out_hbm.at[wid, g], wsem[b])

        def wait_writeout(b):
            pltpu.make_async_copy(outb[b], out_hbm.at[wid, 0], wsem[b]).wait()

        fill_and_drain(0, 0)
        extract(0, 0)
        start_writeout(0, 0)
        fill_and_drain(1, 1)
        extract(1, 1)
        start_writeout(1, 1)

        @pl.loop(2, N_FILLS, step=2)
        def _(g):
            for b in range(2):
                wait_writeout(b)
                fill_and_drain(g + b, b)
                extract(g + b, b)
                start_writeout(g + b, b)

        wait_writeout(0)
        wait_writeout(1)

    return gather


_gather = _make_gather()


def kernel(source, W):
    table = W.reshape(500000, PAIR_W)
    idx = source.reshape(N)
    q = idx >> 1
    hoff = (idx & 1) << 6
    out = _gather(table, q, hoff)
    return out.reshape(SEQ_LEN, BATCH, DIM)


# speedup vs baseline: 1.5461x; 1.5461x over previous
import functools

import jax
import jax.numpy as jnp
from jax import lax
from jax.experimental import pallas as pl
from jax.experimental.pallas import tpu as pltpu
from jax.experimental.pallas import tpu_sc as plsc

SEQ_LEN = 200
BATCH = 1024
DIM = 64
VOCAB = 1000000
N = SEQ_LEN * BATCH
NUM_WORKERS = 32
B_PER_W = N // NUM_WORKERS
CHUNK = 128
GATHERS_PER_FILL = 2
ROWS_PER_FILL = CHUNK * GATHERS_PER_FILL
N_FILLS = B_PER_W // ROWS_PER_FILL


def _make_gather():
    mesh = plsc.VectorSubcoreMesh(core_axis_name="c", subcore_axis_name="s",
                                  num_cores=2)

    @functools.partial(
        pl.kernel,
        mesh=mesh,
        out_type=jax.ShapeDtypeStruct((NUM_WORKERS, N_FILLS, ROWS_PER_FILL,
                                       DIM), jnp.float32),
        scratch_types=[
            pltpu.VMEM((B_PER_W,), jnp.int32),
            pltpu.VMEM((ROWS_PER_FILL, DIM), jnp.float32),
            pltpu.VMEM((ROWS_PER_FILL, DIM), jnp.float32),
            pltpu.SemaphoreType.DMA,
            pltpu.SemaphoreType.DMA,
            pltpu.SemaphoreType.DMA,
            pltpu.SemaphoreType.DMA,
        ],
        compiler_params=pltpu.CompilerParams(use_tc_tiling_on_sc=False,
                                             disable_bounds_checks=True,
                                             needs_layout_passes=False),
    )
    def gather(table_hbm, idx_hbm, out_hbm, idx_v, rows0, rows1,
               gsem0, gsem1, wsem0, wsem1):
        wid = lax.axis_index("s") * 2 + lax.axis_index("c")
        pltpu.sync_copy(idx_hbm.at[pl.ds(wid * B_PER_W, B_PER_W)], idx_v)
        rows = (rows0, rows1)
        gsem = (gsem0, gsem1)
        wsem = (wsem0, wsem1)

        def fill_and_drain(g, b):
            hs = [
                pltpu.async_copy(
                    table_hbm.at[idx_v.at[pl.ds(
                        g * ROWS_PER_FILL + c * CHUNK, CHUNK)]],
                    rows[b].at[pl.ds(c * CHUNK, CHUNK)],
                    gsem[b])
                for c in range(GATHERS_PER_FILL)
            ]
            for h in hs:
                h.wait()

        def start_writeout(g, b):
            pltpu.async_copy(rows[b], out_hbm.at[wid, g], wsem[b])

        def wait_writeout(b):
            pltpu.make_async_copy(rows[b], out_hbm.at[wid, 0], wsem[b]).wait()

        fill_and_drain(0, 0)
        start_writeout(0, 0)
        fill_and_drain(1, 1)
        start_writeout(1, 1)

        @pl.loop(2, N_FILLS - 1, step=2)
        def _(g):
            for b in range(2):
                wait_writeout(b)
                fill_and_drain(g + b, b)
                start_writeout(g + b, b)

        wait_writeout(0)
        fill_and_drain(N_FILLS - 1, 0)
        start_writeout(N_FILLS - 1, 0)

        wait_writeout(0)
        wait_writeout(1)

    return gather


_gather = _make_gather()


def kernel(source, W):
    idx = source.reshape(N)
    out = _gather(W, idx)
    return out.reshape(SEQ_LEN, BATCH, DIM)


# speedup vs baseline: 2.2794x; 1.4743x over previous
import functools

import jax
import jax.numpy as jnp
from jax import lax
from jax.experimental import pallas as pl
from jax.experimental.pallas import tpu as pltpu
from jax.experimental.pallas import tpu_sc as plsc

SEQ_LEN = 200
BATCH = 1024
DIM = 64
PAIR_W = 128
N = SEQ_LEN * BATCH
NUM_WORKERS = 32
B_PER_W = N // NUM_WORKERS
CHUNK = 128
ROWS_PER_FILL = CHUNK
N_FILLS = B_PER_W // ROWS_PER_FILL


def _make_gather():
    mesh = plsc.VectorSubcoreMesh(core_axis_name="c", subcore_axis_name="s",
                                  num_cores=2)

    @functools.partial(
        pl.kernel,
        mesh=mesh,
        out_type=jax.ShapeDtypeStruct((NUM_WORKERS, N_FILLS, ROWS_PER_FILL,
                                       DIM), jnp.float32),
        scratch_types=[
            pltpu.VMEM((B_PER_W,), jnp.int32),
            pltpu.VMEM((B_PER_W,), jnp.int32),
            pltpu.VMEM((ROWS_PER_FILL, PAIR_W), jnp.float32),
            pltpu.VMEM((ROWS_PER_FILL, PAIR_W), jnp.float32),
            pltpu.VMEM((ROWS_PER_FILL, DIM), jnp.float32),
            pltpu.VMEM((ROWS_PER_FILL, DIM), jnp.float32),
            pltpu.SemaphoreType.DMA,
            pltpu.SemaphoreType.DMA,
            pltpu.SemaphoreType.DMA,
            pltpu.SemaphoreType.DMA,
        ],
        compiler_params=pltpu.CompilerParams(use_tc_tiling_on_sc=True,
                                             disable_bounds_checks=True,
                                             needs_layout_passes=False),
    )
    def gather(table_hbm, q_hbm, hoff_hbm, out_hbm, qv, hv,
               pairs0, pairs1, outb0, outb1, gsem0, gsem1, wsem0, wsem1):
        wid = lax.axis_index("s") * 2 + lax.axis_index("c")
        pltpu.sync_copy(q_hbm.at[pl.ds(wid * B_PER_W, B_PER_W)], qv)
        pltpu.sync_copy(hoff_hbm.at[pl.ds(wid * B_PER_W, B_PER_W)], hv)
        pairs = (pairs0, pairs1)
        outb = (outb0, outb1)
        gsem = (gsem0, gsem1)
        wsem = (wsem0, wsem1)

        def fill_and_drain(g, b):
            pltpu.async_copy(
                table_hbm.at[qv.at[pl.ds(g * ROWS_PER_FILL, CHUNK)]],
                pairs[b], gsem[b]).wait()

        def extract(g, b):
            @pl.loop(0, ROWS_PER_FILL // 16)
            def _(i0):
                hvec = hv[pl.ds(g * ROWS_PER_FILL + i0 * 16, 16)]
                for j in range(16):
                    h = hvec[j]
                    row = i0 * 16 + j
                    for k in range(DIM // 16):
                        outb[b][row, pl.ds(16 * k, 16)] = (
                            pairs[b][row, pl.ds(h + 16 * k, 16)])

        def start_writeout(g, b):
            pltpu.async_copy(outb[b], out_hbm.at[wid, g], wsem[b])

        def wait_writeout(b):
            pltpu.make_async_copy(outb[b], out_hbm.at[wid, 0], wsem[b]).wait()

        fill_and_drain(0, 0)
        extract(0, 0)
        start_writeout(0, 0)
        fill_and_drain(1, 1)
        extract(1, 1)
        start_writeout(1, 1)

        @pl.loop(2, N_FILLS, step=2)
        def _(g):
            for b in range(2):
                wait_writeout(b)
                fill_and_drain(g + b, b)
                extract(g + b, b)
                start_writeout(g + b, b)

        wait_writeout(0)
        wait_writeout(1)

    return gather


_gather = _make_gather()


def kernel(source, W):
    table = W.reshape(500000, PAIR_W)
    idx = source.reshape(N)
    q = idx >> 1
    hoff = (idx & 1) << 6
    out = _gather(table, q, hoff)
    return out.reshape(SEQ_LEN, BATCH, DIM)
